# Initial kernel scaffold; baseline (speedup 1.0000x reference)
import jax
import jax.numpy as jnp
from jax.experimental import pallas as pl


def kernel(x, edge_index, edge_attr, u, batch, W1, b1, W2, b2, W3, b3, W4, b4):
    raise NotImplementedError("write your pallas kernel here")


# speedup vs baseline: 2.1590x; 2.1590x over previous
import functools

import jax
import jax.numpy as jnp
from jax import lax
from jax.experimental import pallas as pl
from jax.experimental.pallas import tpu as pltpu
from jax.experimental.pallas import tpu_sc as plsc

N_NODES = 10000
N_PAD = 10240
D_FEAT = 128
D_EDGE = 16
HID = 256
D_OUT = 128
NC = 2
NS = 16
NW = NC * NS
CH = 80


def _sc_mesh():
    return plsc.VectorSubcoreMesh(
        core_axis_name="c", subcore_axis_name="s", num_cores=NC, num_subcores=NS
    )


def _fill(buf, n, value):
    @pl.loop(0, n)
    def _(i):
        @pl.loop(0, D_FEAT // 16)
        def _(j):
            buf[i, pl.ds(j * 16, 16)] = jnp.full((16,), value, jnp.float32)


def _sc_count(col):
    e = col.shape[0]
    per_t = e // NW
    n_ch = per_t // CH
    rpt = N_PAD // NS

    @functools.partial(
        pl.kernel,
        out_type=(
            jax.ShapeDtypeStruct((N_PAD, D_FEAT), jnp.float32),
            jax.ShapeDtypeStruct((N_PAD, D_FEAT), jnp.float32),
        ),
        mesh=_sc_mesh(),
        scratch_types=[
            pltpu.VMEM((CH,), jnp.int32),
            pltpu.VMEM((CH, D_FEAT), jnp.float32),
            pltpu.VMEM_SHARED((N_PAD, D_FEAT), jnp.float32),
        ],
    )
    def k(col_hbm, c0_hbm, c1_hbm, idx_v, buf, acc_sh):
        cid = lax.axis_index("c")
        sid = lax.axis_index("s")
        row0 = sid * rpt

        _fill(buf, CH, 0.0)

        @pl.loop(0, rpt // CH)
        def _(j):
            pltpu.sync_copy(buf, acc_sh.at[pl.ds(row0 + j * CH, CH)])

        _fill(buf, CH, 1.0)
        plsc.subcore_barrier()

        @pl.loop(0, n_ch)
        def _(i):
            off = (cid * NS + sid) * per_t + i * CH
            pltpu.sync_copy(col_hbm.at[pl.ds(off, CH)], idx_v)
            pltpu.sync_copy(buf, acc_sh.at[idx_v], add=True)

        plsc.subcore_barrier()

        def write(c_hbm):
            @pl.loop(0, rpt // CH)
            def _(j):
                r = row0 + j * CH
                pltpu.sync_copy(acc_sh.at[pl.ds(r, CH)], buf)
                pltpu.sync_copy(buf, c_hbm.at[pl.ds(r, CH)])

        @pl.when(cid == 0)
        def _():
            write(c0_hbm)

        @pl.when(cid == 1)
        def _():
            write(c1_hbm)

    return k(col)


def _sc_gather(x, row):
    e = row.shape[0]
    per_w = e // NW
    n_ch = per_w // CH

    @functools.partial(
        pl.kernel,
        out_type=jax.ShapeDtypeStruct((e, D_FEAT), jnp.float32),
        mesh=_sc_mesh(),
        scratch_types=[
            pltpu.VMEM((CH,), jnp.int32),
            pltpu.VMEM((CH, D_FEAT), jnp.float32),
            pltpu.SemaphoreType.DMA,
        ],
    )
    def k(x_hbm, row_hbm, out_hbm, idx_v, rows_v, sem):
        wid = lax.axis_index("s") * NC + lax.axis_index("c")
        base = wid * per_w

        @pl.loop(0, n_ch)
        def _(i):
            off = base + i * CH
            pltpu.sync_copy(row_hbm.at[pl.ds(off, CH)], idx_v)
            pltpu.async_copy(x_hbm.at[idx_v], rows_v, sem).wait()
            pltpu.sync_copy(rows_v, out_hbm.at[pl.ds(off, CH)])

    return k(x, row)


def _mlp1_body(xg_ref, ea_ref, w1a_ref, w1b_ref, b1_ref, h0_ref, h1_ref):
    g = jnp.dot(xg_ref[...], w1a_ref[...], preferred_element_type=jnp.float32)
    g = g + jnp.dot(ea_ref[...], w1b_ref[...], preferred_element_type=jnp.float32)
    g = g + b1_ref[...]
    g = jnp.where(g > 0, g, jnp.exp(jnp.minimum(g, 0.0)) - 1.0)
    h0_ref[...] = g[:, :D_FEAT]
    h1_ref[...] = g[:, D_FEAT:]


def _tc_mlp1(xg, ea, w1a, w1b, b1, blk=512):
    e = xg.shape[0]
    grid = (e // blk,)
    return pl.pallas_call(
        _mlp1_body,
        grid=grid,
        in_specs=[
            pl.BlockSpec((blk, D_FEAT), lambda i: (i, 0)),
            pl.BlockSpec((blk, D_EDGE), lambda i: (i, 0)),
            pl.BlockSpec((D_FEAT, HID), lambda i: (0, 0)),
            pl.BlockSpec((D_EDGE, HID), lambda i: (0, 0)),
            pl.BlockSpec((1, HID), lambda i: (0, 0)),
        ],
        out_specs=[
            pl.BlockSpec((blk, D_FEAT), lambda i: (i, 0)),
            pl.BlockSpec((blk, D_FEAT), lambda i: (i, 0)),
        ],
        out_shape=[
            jax.ShapeDtypeStruct((e, D_FEAT), jnp.float32),
            jax.ShapeDtypeStruct((e, D_FEAT), jnp.float32),
        ],
    )(xg, ea, w1a, w1b, b1)


def _sc_scatter(h0, h1, col):
    e = col.shape[0]
    per_t = e // NS
    n_ch = per_t // CH
    rpt = N_PAD // NS

    @functools.partial(
        pl.kernel,
        out_type=(
            jax.ShapeDtypeStruct((N_PAD, D_FEAT), jnp.float32),
            jax.ShapeDtypeStruct((N_PAD, D_FEAT), jnp.float32),
        ),
        mesh=_sc_mesh(),
        scratch_types=[
            pltpu.VMEM((CH,), jnp.int32),
            pltpu.VMEM((CH, D_FEAT), jnp.float32),
            pltpu.VMEM_SHARED((N_PAD, D_FEAT), jnp.float32),
        ],
    )
    def k(h0_hbm, h1_hbm, col_hbm, s0_hbm, s1_hbm, idx_v, hbuf, acc_sh):
        cid = lax.axis_index("c")
        sid = lax.axis_index("s")
        row0 = sid * rpt

        _fill(hbuf, CH, 0.0)

        @pl.loop(0, rpt // CH)
        def _(j):
            pltpu.sync_copy(hbuf, acc_sh.at[pl.ds(row0 + j * CH, CH)])

        plsc.subcore_barrier()

        def scatter(h_hbm):
            @pl.loop(0, n_ch)
            def _(i):
                off = sid * per_t + i * CH
                pltpu.sync_copy(col_hbm.at[pl.ds(off, CH)], idx_v)
                pltpu.sync_copy(h_hbm.at[pl.ds(off, CH)], hbuf)
                pltpu.sync_copy(hbuf, acc_sh.at[idx_v], add=True)

        @pl.when(cid == 0)
        def _():
            scatter(h0_hbm)

        @pl.when(cid == 1)
        def _():
            scatter(h1_hbm)

        plsc.subcore_barrier()

        def write(s_hbm):
            @pl.loop(0, rpt // CH)
            def _(j):
                r = row0 + j * CH
                pltpu.sync_copy(acc_sh.at[pl.ds(r, CH)], hbuf)
                pltpu.sync_copy(hbuf, s_hbm.at[pl.ds(r, CH)])

        @pl.when(cid == 0)
        def _():
            write(s0_hbm)

        @pl.when(cid == 1)
        def _():
            write(s1_hbm)

    return k(h0, h1, col)


def _mlp2_body(x_ref, s0_ref, s1_ref, c0_ref, c1_ref, w2a_ref, w2b_ref, b2_ref,
               w3x_ref, w3m_ref, b3_ref, w4_ref, b4_ref, o_ref):
    c = c0_ref[:, 0:1] + c1_ref[:, 0:1]
    r = 1.0 / jnp.maximum(c, 1.0)
    has = jnp.where(c > 0, 1.0, 0.0)
    mean = (
        jnp.dot(s0_ref[...] * r, w2a_ref[...], preferred_element_type=jnp.float32)
        + jnp.dot(s1_ref[...] * r, w2b_ref[...], preferred_element_type=jnp.float32)
        + b2_ref[...] * has
    )
    h2 = (
        jnp.dot(x_ref[...], w3x_ref[...], preferred_element_type=jnp.float32)
        + jnp.dot(mean, w3m_ref[...], preferred_element_type=jnp.float32)
        + b3_ref[...]
    )
    h2 = jnp.where(h2 > 0, h2, jnp.exp(jnp.minimum(h2, 0.0)) - 1.0)
    o_ref[...] = jnp.dot(h2, w4_ref[...], preferred_element_type=jnp.float32) + b4_ref[...]


def _tc_mlp2(x, s0, s1, c0, c1, w2a, w2b, b2, w3x, w3m, b3, w4, b4, blk=400):
    n = x.shape[0]
    grid = (n // blk,)
    return pl.pallas_call(
        _mlp2_body,
        grid=grid,
        in_specs=[
            pl.BlockSpec((blk, D_FEAT), lambda i: (i, 0)),
            pl.BlockSpec((blk, D_FEAT), lambda i: (i, 0)),
            pl.BlockSpec((blk, D_FEAT), lambda i: (i, 0)),
            pl.BlockSpec((blk, D_FEAT), lambda i: (i, 0)),
            pl.BlockSpec((blk, D_FEAT), lambda i: (i, 0)),
            pl.BlockSpec((D_FEAT, HID), lambda i: (0, 0)),
            pl.BlockSpec((D_FEAT, HID), lambda i: (0, 0)),
            pl.BlockSpec((1, HID), lambda i: (0, 0)),
            pl.BlockSpec((D_FEAT, HID), lambda i: (0, 0)),
            pl.BlockSpec((HID, HID), lambda i: (0, 0)),
            pl.BlockSpec((1, HID), lambda i: (0, 0)),
            pl.BlockSpec((HID, D_OUT), lambda i: (0, 0)),
            pl.BlockSpec((1, D_OUT), lambda i: (0, 0)),
        ],
        out_specs=pl.BlockSpec((blk, D_OUT), lambda i: (i, 0)),
        out_shape=jax.ShapeDtypeStruct((n, D_OUT), jnp.float32),
    )(x, s0, s1, c0, c1, w2a, w2b, b2, w3x, w3m, b3, w4, b4)


def kernel(x, edge_index, edge_attr, u, batch, W1, b1, W2, b2, W3, b3, W4, b4):
    del u, batch
    row = edge_index[0]
    col = edge_index[1]

    c0, c1 = _sc_count(col)
    xg = _sc_gather(x, row)
    h0, h1 = _tc_mlp1(
        xg, edge_attr, W1[:D_FEAT], W1[D_FEAT:], b1.reshape(1, HID)
    )
    s0, s1 = _sc_scatter(h0, h1, col)
    out = _tc_mlp2(
        x,
        s0[:N_NODES],
        s1[:N_NODES],
        c0[:N_NODES],
        c1[:N_NODES],
        W2[:D_FEAT],
        W2[D_FEAT:],
        b2.reshape(1, HID),
        W3[:D_FEAT],
        W3[D_FEAT:],
        b3.reshape(1, HID),
        W4,
        b4.reshape(1, D_OUT),
    )
    return out


# speedup vs baseline: 2.8055x; 1.2994x over previous
import functools

import jax
import jax.numpy as jnp
from jax import lax
from jax.experimental import pallas as pl
from jax.experimental.pallas import tpu as pltpu
from jax.experimental.pallas import tpu_sc as plsc

N_NODES = 10000
N_PAD = 10240
D_FEAT = 128
D_EDGE = 16
HID = 256
D_OUT = 128
NC = 2
NS = 16
NW = NC * NS
CH = 80


def _sc_mesh():
    return plsc.VectorSubcoreMesh(
        core_axis_name="c", subcore_axis_name="s", num_cores=NC, num_subcores=NS
    )


def _fill(buf, n, value):
    @pl.loop(0, n)
    def _(i):
        @pl.loop(0, D_FEAT // 16)
        def _(j):
            buf[i, pl.ds(j * 16, 16)] = jnp.full((16,), value, jnp.float32)


def _sc_count(col2d):
    n_rows = col2d.shape[0]
    per_t = n_rows // NW
    rpt = N_PAD // NS
    nb = 5
    assert per_t % nb == 0

    @functools.partial(
        pl.kernel,
        out_type=(
            jax.ShapeDtypeStruct((N_PAD, D_FEAT), jnp.float32),
            jax.ShapeDtypeStruct((N_PAD, D_FEAT), jnp.float32),
        ),
        mesh=_sc_mesh(),
        scratch_types=[
            pltpu.VMEM((per_t, 1, CH), jnp.int32),
            pltpu.VMEM((CH, D_FEAT), jnp.float32),
            pltpu.VMEM_SHARED((N_PAD, D_FEAT), jnp.float32),
            pltpu.SemaphoreType.DMA((nb,)),
        ],
    )
    def k(col_hbm, c0_hbm, c1_hbm, idx2d, buf, acc_sh, sems):
        cid = lax.axis_index("c")
        sid = lax.axis_index("s")
        row0 = sid * rpt

        _fill(buf, CH, 0.0)

        @pl.loop(0, rpt // CH)
        def _(j):
            pltpu.sync_copy(buf, acc_sh.at[pl.ds(row0 + j * CH, CH)])

        pltpu.sync_copy(col_hbm.at[pl.ds((cid * NS + sid) * per_t, per_t)], idx2d)
        _fill(buf, CH, 1.0)
        plsc.subcore_barrier()

        @pl.loop(0, per_t, step=nb)
        def _(i):
            ds = [
                pltpu.async_copy(buf, acc_sh.at[idx2d.at[i + b, 0]], sems.at[b],
                                 add=True)
                for b in range(nb)
            ]
            for d in ds:
                d.wait()

        plsc.subcore_barrier()

        def write(c_hbm):
            @pl.loop(0, rpt // CH)
            def _(j):
                r = row0 + j * CH
                pltpu.sync_copy(acc_sh.at[pl.ds(r, CH)], buf)
                pltpu.sync_copy(buf, c_hbm.at[pl.ds(r, CH)])

        @pl.when(cid == 0)
        def _():
            write(c0_hbm)

        @pl.when(cid == 1)
        def _():
            write(c1_hbm)

    return k(col2d)


def _sc_gather(x, row):
    e = row.shape[0]
    per_w = e // NW
    n_ch = per_w // CH
    nb = 5
    assert n_ch % nb == 0

    @functools.partial(
        pl.kernel,
        out_type=jax.ShapeDtypeStruct((e, D_FEAT), jnp.float32),
        mesh=_sc_mesh(),
        scratch_types=[
            pltpu.VMEM((per_w,), jnp.int32),
            pltpu.VMEM((nb, CH, D_FEAT), jnp.float32),
            pltpu.SemaphoreType.DMA((nb,)),
            pltpu.SemaphoreType.DMA((nb,)),
        ],
    )
    def k(x_hbm, row_hbm, out_hbm, idx_all, rows_v, gsems, ssems):
        wid = lax.axis_index("s") * NC + lax.axis_index("c")
        base = wid * per_w

        pltpu.sync_copy(row_hbm.at[pl.ds(base, per_w)], idx_all)

        @pl.loop(0, n_ch, step=nb)
        def _(i):
            gds = [
                pltpu.async_copy(
                    x_hbm.at[idx_all.at[pl.ds((i + b) * CH, CH)]],
                    rows_v.at[b], gsems.at[b])
                for b in range(nb)
            ]
            sds = []
            for b in range(nb):
                gds[b].wait()
                sds.append(pltpu.async_copy(
                    rows_v.at[b],
                    out_hbm.at[pl.ds(base + (i + b) * CH, CH)],
                    ssems.at[b]))
            for d in sds:
                d.wait()

    return k(x, row)


def _mlp1_body(xg_ref, ea_ref, w1a_ref, w1b_ref, b1_ref, h0_ref, h1_ref):
    g = jnp.dot(xg_ref[...], w1a_ref[...], preferred_element_type=jnp.float32)
    g = g + jnp.dot(ea_ref[...], w1b_ref[...], preferred_element_type=jnp.float32)
    g = g + b1_ref[...]
    g = jnp.where(g > 0, g, jnp.exp(jnp.minimum(g, 0.0)) - 1.0)
    h0_ref[...] = g[:, :D_FEAT]
    h1_ref[...] = g[:, D_FEAT:]


def _tc_mlp1(xg, ea, w1a, w1b, b1, blk=512):
    e = xg.shape[0]
    grid = (e // blk,)
    return pl.pallas_call(
        _mlp1_body,
        grid=grid,
        in_specs=[
            pl.BlockSpec((blk, D_FEAT), lambda i: (i, 0)),
            pl.BlockSpec((blk, D_EDGE), lambda i: (i, 0)),
            pl.BlockSpec((D_FEAT, HID), lambda i: (0, 0)),
            pl.BlockSpec((D_EDGE, HID), lambda i: (0, 0)),
            pl.BlockSpec((1, HID), lambda i: (0, 0)),
        ],
        out_specs=[
            pl.BlockSpec((blk, D_FEAT), lambda i: (i, 0)),
            pl.BlockSpec((blk, D_FEAT), lambda i: (i, 0)),
        ],
        out_shape=[
            jax.ShapeDtypeStruct((e, D_FEAT), jnp.float32),
            jax.ShapeDtypeStruct((e, D_FEAT), jnp.float32),
        ],
    )(xg, ea, w1a, w1b, b1)


def _sc_scatter(h0, h1, col2d):
    n_rows = col2d.shape[0]
    per_t = n_rows // NS
    rpt = N_PAD // NS
    nb = 4
    ib = 10
    assert per_t % ib == 0

    @functools.partial(
        pl.kernel,
        out_type=(
            jax.ShapeDtypeStruct((N_PAD, D_FEAT), jnp.float32),
            jax.ShapeDtypeStruct((N_PAD, D_FEAT), jnp.float32),
        ),
        mesh=_sc_mesh(),
        scratch_types=[
            pltpu.VMEM((ib, 1, CH), jnp.int32),
            pltpu.VMEM((nb, CH, D_FEAT), jnp.float32),
            pltpu.VMEM_SHARED((N_PAD, D_FEAT), jnp.float32),
            pltpu.SemaphoreType.DMA((nb,)),
            pltpu.SemaphoreType.DMA((nb,)),
        ],
    )
    def k(h0_hbm, h1_hbm, col_hbm, s0_hbm, s1_hbm,
          idx2d, hbufs, acc_sh, lsems, asems):
        cid = lax.axis_index("c")
        sid = lax.axis_index("s")
        row0 = sid * rpt

        _fill(hbufs.at[0], CH, 0.0)

        @pl.loop(0, rpt // CH)
        def _(j):
            pltpu.sync_copy(hbufs.at[0], acc_sh.at[pl.ds(row0 + j * CH, CH)])

        plsc.subcore_barrier()

        def scatter(h_hbm):
            @pl.loop(0, per_t, step=ib)
            def _(i):
                pltpu.sync_copy(col_hbm.at[pl.ds(sid * per_t + i, ib)], idx2d)
                for w0 in range(0, ib, nb):
                    wave = range(w0, min(w0 + nb, ib))
                    lds = [
                        pltpu.async_copy(
                            h_hbm.at[pl.ds((sid * per_t + i + c) * CH, CH)],
                            hbufs.at[c - w0], lsems.at[c - w0])
                        for c in wave
                    ]
                    ads = []
                    for c in wave:
                        lds[c - w0].wait()
                        ads.append(pltpu.async_copy(
                            hbufs.at[c - w0], acc_sh.at[idx2d.at[c, 0]],
                            asems.at[c - w0], add=True))
                    for d in ads:
                        d.wait()

        @pl.when(cid == 0)
        def _():
            scatter(h0_hbm)

        @pl.when(cid == 1)
        def _():
            scatter(h1_hbm)

        plsc.subcore_barrier()

        def write(s_hbm):
            @pl.loop(0, rpt // CH)
            def _(j):
                r = row0 + j * CH
                pltpu.sync_copy(acc_sh.at[pl.ds(r, CH)], hbufs.at[0])
                pltpu.sync_copy(hbufs.at[0], s_hbm.at[pl.ds(r, CH)])

        @pl.when(cid == 0)
        def _():
            write(s0_hbm)

        @pl.when(cid == 1)
        def _():
            write(s1_hbm)

    return k(h0, h1, col2d)


def _mlp2_body(x_ref, s0_ref, s1_ref, c0_ref, c1_ref, w2a_ref, w2b_ref, b2_ref,
               w3x_ref, w3m_ref, b3_ref, w4_ref, b4_ref, o_ref):
    c = c0_ref[:, 0:1] + c1_ref[:, 0:1]
    r = 1.0 / jnp.maximum(c, 1.0)
    has = jnp.where(c > 0, 1.0, 0.0)
    mean = (
        jnp.dot(s0_ref[...] * r, w2a_ref[...], preferred_element_type=jnp.float32)
        + jnp.dot(s1_ref[...] * r, w2b_ref[...], preferred_element_type=jnp.float32)
        + b2_ref[...] * has
    )
    h2 = (
        jnp.dot(x_ref[...], w3x_ref[...], preferred_element_type=jnp.float32)
        + jnp.dot(mean, w3m_ref[...], preferred_element_type=jnp.float32)
        + b3_ref[...]
    )
    h2 = jnp.where(h2 > 0, h2, jnp.exp(jnp.minimum(h2, 0.0)) - 1.0)
    o_ref[...] = jnp.dot(h2, w4_ref[...], preferred_element_type=jnp.float32) + b4_ref[...]


def _tc_mlp2(x, s0, s1, c0, c1, w2a, w2b, b2, w3x, w3m, b3, w4, b4, blk=400):
    n = x.shape[0]
    grid = (n // blk,)
    return pl.pallas_call(
        _mlp2_body,
        grid=grid,
        in_specs=[
            pl.BlockSpec((blk, D_FEAT), lambda i: (i, 0)),
            pl.BlockSpec((blk, D_FEAT), lambda i: (i, 0)),
            pl.BlockSpec((blk, D_FEAT), lambda i: (i, 0)),
            pl.BlockSpec((blk, D_FEAT), lambda i: (i, 0)),
            pl.BlockSpec((blk, D_FEAT), lambda i: (i, 0)),
            pl.BlockSpec((D_FEAT, HID), lambda i: (0, 0)),
            pl.BlockSpec((D_FEAT, HID), lambda i: (0, 0)),
            pl.BlockSpec((1, HID), lambda i: (0, 0)),
            pl.BlockSpec((D_FEAT, HID), lambda i: (0, 0)),
            pl.BlockSpec((HID, HID), lambda i: (0, 0)),
            pl.BlockSpec((1, HID), lambda i: (0, 0)),
            pl.BlockSpec((HID, D_OUT), lambda i: (0, 0)),
            pl.BlockSpec((1, D_OUT), lambda i: (0, 0)),
        ],
        out_specs=pl.BlockSpec((blk, D_OUT), lambda i: (i, 0)),
        out_shape=jax.ShapeDtypeStruct((n, D_OUT), jnp.float32),
    )(x, s0, s1, c0, c1, w2a, w2b, b2, w3x, w3m, b3, w4, b4)


def kernel(x, edge_index, edge_attr, u, batch, W1, b1, W2, b2, W3, b3, W4, b4):
    del u, batch
    row = edge_index[0]
    col2d = edge_index[1].reshape(-1, 1, CH)

    c0, c1 = _sc_count(col2d)
    xg = _sc_gather(x, row)
    h0, h1 = _tc_mlp1(
        xg, edge_attr, W1[:D_FEAT], W1[D_FEAT:], b1.reshape(1, HID)
    )
    s0, s1 = _sc_scatter(h0, h1, col2d)
    out = _tc_mlp2(
        x,
        s0[:N_NODES],
        s1[:N_NODES],
        c0[:N_NODES],
        c1[:N_NODES],
        W2[:D_FEAT],
        W2[D_FEAT:],
        b2.reshape(1, HID),
        W3[:D_FEAT],
        W3[D_FEAT:],
        b3.reshape(1, HID),
        W4,
        b4.reshape(1, D_OUT),
    )
    return out


# speedup vs baseline: 3.0559x; 1.0893x over previous
import functools

import jax
import jax.numpy as jnp
from jax import lax
from jax.experimental import pallas as pl
from jax.experimental.pallas import tpu as pltpu
from jax.experimental.pallas import tpu_sc as plsc

N_NODES = 10000
N_PAD = 10240
D_FEAT = 128
D_EDGE = 16
HID = 256
D_OUT = 128
NC = 2
NS = 16
NW = NC * NS
CH = 80


def _sc_mesh():
    return plsc.VectorSubcoreMesh(
        core_axis_name="c", subcore_axis_name="s", num_cores=NC, num_subcores=NS
    )


def _fill(buf, n, value):
    @pl.loop(0, n)
    def _(i):
        @pl.loop(0, D_FEAT // 16)
        def _(j):
            buf[i, pl.ds(j * 16, 16)] = jnp.full((16,), value, jnp.float32)


def _sc_count(col2d):
    n_rows = col2d.shape[0]
    per_t = n_rows // NW
    rpt = N_PAD // NS
    nb = 5
    assert per_t % nb == 0

    @functools.partial(
        pl.kernel,
        out_type=(
            jax.ShapeDtypeStruct((N_PAD, D_FEAT), jnp.float32),
            jax.ShapeDtypeStruct((N_PAD, D_FEAT), jnp.float32),
        ),
        mesh=_sc_mesh(),
        scratch_types=[
            pltpu.VMEM((per_t, 1, CH), jnp.int32),
            pltpu.VMEM((CH, D_FEAT), jnp.float32),
            pltpu.VMEM_SHARED((N_PAD, D_FEAT), jnp.float32),
            pltpu.SemaphoreType.DMA((nb,)),
        ],
    )
    def k(col_hbm, c0_hbm, c1_hbm, idx2d, buf, acc_sh, sems):
        cid = lax.axis_index("c")
        sid = lax.axis_index("s")
        row0 = sid * rpt

        _fill(buf, CH, 0.0)

        @pl.loop(0, rpt // CH)
        def _(j):
            pltpu.sync_copy(buf, acc_sh.at[pl.ds(row0 + j * CH, CH)])

        pltpu.sync_copy(col_hbm.at[pl.ds((cid * NS + sid) * per_t, per_t)], idx2d)
        _fill(buf, CH, 1.0)
        plsc.subcore_barrier()

        @pl.loop(0, per_t, step=25)
        def _(i):
            ds = {}
            for c in range(25):
                if c >= nb:
                    ds[c - nb].wait()
                ds[c] = pltpu.async_copy(
                    buf, acc_sh.at[idx2d.at[i + c, 0]], sems.at[c % nb],
                    add=True)
            for c in range(25 - nb, 25):
                ds[c].wait()

        plsc.subcore_barrier()

        def write(c_hbm):
            @pl.loop(0, rpt // CH)
            def _(j):
                r = row0 + j * CH
                pltpu.sync_copy(acc_sh.at[pl.ds(r, CH)], buf)
                pltpu.sync_copy(buf, c_hbm.at[pl.ds(r, CH)])

        @pl.when(cid == 0)
        def _():
            write(c0_hbm)

        @pl.when(cid == 1)
        def _():
            write(c1_hbm)

    return k(col2d)


def _sc_gather(x, row):
    e = row.shape[0]
    per_w = e // NW
    n_ch = per_w // CH
    nb = 8
    assert n_ch % 25 == 0

    @functools.partial(
        pl.kernel,
        out_type=jax.ShapeDtypeStruct((e, D_FEAT), jnp.float32),
        mesh=_sc_mesh(),
        scratch_types=[
            pltpu.VMEM((per_w,), jnp.int32),
            pltpu.VMEM((nb, CH, D_FEAT), jnp.float32),
            pltpu.SemaphoreType.DMA((nb,)),
            pltpu.SemaphoreType.DMA((nb,)),
        ],
    )
    def k(x_hbm, row_hbm, out_hbm, idx_all, rows_v, gsems, ssems):
        wid = lax.axis_index("s") * NC + lax.axis_index("c")
        base = wid * per_w

        pltpu.sync_copy(row_hbm.at[pl.ds(base, per_w)], idx_all)

        @pl.loop(0, n_ch, step=25)
        def _(i):
            gds = {}
            sds = {}

            def put(c):
                gds[c].wait()
                sds[c] = pltpu.async_copy(
                    rows_v.at[c % nb],
                    out_hbm.at[pl.ds(base + (i + c) * CH, CH)],
                    ssems.at[c % nb])

            for c in range(25):
                if c >= nb:
                    sds[c - nb].wait()
                gds[c] = pltpu.async_copy(
                    x_hbm.at[idx_all.at[pl.ds((i + c) * CH, CH)]],
                    rows_v.at[c % nb], gsems.at[c % nb])
                if c >= nb - 1:
                    put(c - nb + 1)
            for c in range(25 - nb + 1, 25):
                put(c)
            for c in range(25 - nb, 25):
                sds[c].wait()

    return k(x, row)


def _mlp1_body(xg_ref, ea_ref, w1a_ref, w1b_ref, b1_ref, h0_ref, h1_ref):
    g = jnp.dot(xg_ref[...], w1a_ref[...], preferred_element_type=jnp.float32)
    g = g + jnp.dot(ea_ref[...], w1b_ref[...], preferred_element_type=jnp.float32)
    g = g + b1_ref[...]
    g = jnp.where(g > 0, g, jnp.exp(jnp.minimum(g, 0.0)) - 1.0)
    h0_ref[...] = g[:, :D_FEAT]
    h1_ref[...] = g[:, D_FEAT:]


def _tc_mlp1(xg, ea, w1a, w1b, b1, blk=512):
    e = xg.shape[0]
    grid = (e // blk,)
    return pl.pallas_call(
        _mlp1_body,
        grid=grid,
        in_specs=[
            pl.BlockSpec((blk, D_FEAT), lambda i: (i, 0)),
            pl.BlockSpec((blk, D_EDGE), lambda i: (i, 0)),
            pl.BlockSpec((D_FEAT, HID), lambda i: (0, 0)),
            pl.BlockSpec((D_EDGE, HID), lambda i: (0, 0)),
            pl.BlockSpec((1, HID), lambda i: (0, 0)),
        ],
        out_specs=[
            pl.BlockSpec((blk, D_FEAT), lambda i: (i, 0)),
            pl.BlockSpec((blk, D_FEAT), lambda i: (i, 0)),
        ],
        out_shape=[
            jax.ShapeDtypeStruct((e, D_FEAT), jnp.float32),
            jax.ShapeDtypeStruct((e, D_FEAT), jnp.float32),
        ],
    )(xg, ea, w1a, w1b, b1)


def _sc_scatter(h0, h1, col2d):
    n_rows = col2d.shape[0]
    per_t = n_rows // NS
    rpt = N_PAD // NS
    nb = 4
    ib = 25
    assert per_t % ib == 0

    @functools.partial(
        pl.kernel,
        out_type=(
            jax.ShapeDtypeStruct((N_PAD, D_FEAT), jnp.float32),
            jax.ShapeDtypeStruct((N_PAD, D_FEAT), jnp.float32),
        ),
        mesh=_sc_mesh(),
        scratch_types=[
            pltpu.VMEM((ib, 1, CH), jnp.int32),
            pltpu.VMEM((nb, CH, D_FEAT), jnp.float32),
            pltpu.VMEM_SHARED((N_PAD, D_FEAT), jnp.float32),
            pltpu.SemaphoreType.DMA((nb,)),
            pltpu.SemaphoreType.DMA((nb,)),
        ],
    )
    def k(h0_hbm, h1_hbm, col_hbm, s0_hbm, s1_hbm,
          idx2d, hbufs, acc_sh, lsems, asems):
        cid = lax.axis_index("c")
        sid = lax.axis_index("s")
        row0 = sid * rpt

        _fill(hbufs.at[0], CH, 0.0)

        @pl.loop(0, rpt // CH)
        def _(j):
            pltpu.sync_copy(hbufs.at[0], acc_sh.at[pl.ds(row0 + j * CH, CH)])

        plsc.subcore_barrier()

        def scatter(h_hbm):
            @pl.loop(0, per_t, step=ib)
            def _(i):
                pltpu.sync_copy(col_hbm.at[pl.ds(sid * per_t + i, ib)], idx2d)
                lds = {}
                ads = {}

                def add(c):
                    lds[c].wait()
                    ads[c] = pltpu.async_copy(
                        hbufs.at[c % nb], acc_sh.at[idx2d.at[c, 0]],
                        asems.at[c % nb], add=True)

                for c in range(ib):
                    if c >= nb:
                        ads[c - nb].wait()
                    lds[c] = pltpu.async_copy(
                        h_hbm.at[pl.ds((sid * per_t + i + c) * CH, CH)],
                        hbufs.at[c % nb], lsems.at[c % nb])
                    if c >= 2:
                        add(c - 2)
                for c in range(ib - 2, ib):
                    add(c)
                for c in range(ib - nb, ib):
                    ads[c].wait()

        @pl.when(cid == 0)
        def _():
            scatter(h0_hbm)

        @pl.when(cid == 1)
        def _():
            scatter(h1_hbm)

        plsc.subcore_barrier()

        def write(s_hbm):
            @pl.loop(0, rpt // CH)
            def _(j):
                r = row0 + j * CH
                pltpu.sync_copy(acc_sh.at[pl.ds(r, CH)], hbufs.at[0])
                pltpu.sync_copy(hbufs.at[0], s_hbm.at[pl.ds(r, CH)])

        @pl.when(cid == 0)
        def _():
            write(s0_hbm)

        @pl.when(cid == 1)
        def _():
            write(s1_hbm)

    return k(h0, h1, col2d)


def _mlp2_body(x_ref, s0_ref, s1_ref, c0_ref, c1_ref, w2a_ref, w2b_ref, b2_ref,
               w3x_ref, w3m_ref, b3_ref, w4_ref, b4_ref, o_ref):
    c = c0_ref[:, 0:1] + c1_ref[:, 0:1]
    r = 1.0 / jnp.maximum(c, 1.0)
    has = jnp.where(c > 0, 1.0, 0.0)
    mean = (
        jnp.dot(s0_ref[...] * r, w2a_ref[...], preferred_element_type=jnp.float32)
        + jnp.dot(s1_ref[...] * r, w2b_ref[...], preferred_element_type=jnp.float32)
        + b2_ref[...] * has
    )
    h2 = (
        jnp.dot(x_ref[...], w3x_ref[...], preferred_element_type=jnp.float32)
        + jnp.dot(mean, w3m_ref[...], preferred_element_type=jnp.float32)
        + b3_ref[...]
    )
    h2 = jnp.where(h2 > 0, h2, jnp.exp(jnp.minimum(h2, 0.0)) - 1.0)
    o_ref[...] = jnp.dot(h2, w4_ref[...], preferred_element_type=jnp.float32) + b4_ref[...]


def _tc_mlp2(x, s0, s1, c0, c1, w2a, w2b, b2, w3x, w3m, b3, w4, b4, blk=400):
    n = x.shape[0]
    grid = (n // blk,)
    return pl.pallas_call(
        _mlp2_body,
        grid=grid,
        in_specs=[
            pl.BlockSpec((blk, D_FEAT), lambda i: (i, 0)),
            pl.BlockSpec((blk, D_FEAT), lambda i: (i, 0)),
            pl.BlockSpec((blk, D_FEAT), lambda i: (i, 0)),
            pl.BlockSpec((blk, D_FEAT), lambda i: (i, 0)),
            pl.BlockSpec((blk, D_FEAT), lambda i: (i, 0)),
            pl.BlockSpec((D_FEAT, HID), lambda i: (0, 0)),
            pl.BlockSpec((D_FEAT, HID), lambda i: (0, 0)),
            pl.BlockSpec((1, HID), lambda i: (0, 0)),
            pl.BlockSpec((D_FEAT, HID), lambda i: (0, 0)),
            pl.BlockSpec((HID, HID), lambda i: (0, 0)),
            pl.BlockSpec((1, HID), lambda i: (0, 0)),
            pl.BlockSpec((HID, D_OUT), lambda i: (0, 0)),
            pl.BlockSpec((1, D_OUT), lambda i: (0, 0)),
        ],
        out_specs=pl.BlockSpec((blk, D_OUT), lambda i: (i, 0)),
        out_shape=jax.ShapeDtypeStruct((n, D_OUT), jnp.float32),
    )(x, s0, s1, c0, c1, w2a, w2b, b2, w3x, w3m, b3, w4, b4)


def kernel(x, edge_index, edge_attr, u, batch, W1, b1, W2, b2, W3, b3, W4, b4):
    del u, batch
    row = edge_index[0]
    col2d = edge_index[1].reshape(-1, 1, CH)

    c0, c1 = _sc_count(col2d)
    xg = _sc_gather(x, row)
    h0, h1 = _tc_mlp1(
        xg, edge_attr, W1[:D_FEAT], W1[D_FEAT:], b1.reshape(1, HID)
    )
    s0, s1 = _sc_scatter(h0, h1, col2d)
    out = _tc_mlp2(
        x,
        s0[:N_NODES],
        s1[:N_NODES],
        c0[:N_NODES],
        c1[:N_NODES],
        W2[:D_FEAT],
        W2[D_FEAT:],
        b2.reshape(1, HID),
        W3[:D_FEAT],
        W3[D_FEAT:],
        b3.reshape(1, HID),
        W4,
        b4.reshape(1, D_OUT),
    )
    return out
